# baseline (speedup 1.0000x reference)
import functools

import jax
import jax.numpy as jnp
from jax import lax
from jax.experimental import pallas as pl
from jax.experimental.pallas import tpu as pltpu
from jax.experimental.pallas import tpu_sc as plsc

BATCH = 16384
EMB_DIM = 128
NUM_EMB = 16

NC = 2
NS = 16
NW = NC * NS
TOTAL = BATCH * 16
ROWS_W = TOTAL // NW
CHUNK = 128
NCHUNK = ROWS_W // CHUNK
NBUF = 4
IDX_ROWS = ROWS_W // CHUNK


def _body(table_hbm, x_hbm, out_hbm, xv, idx2d, bufs, gsem, *wsems):
    cid = lax.axis_index("c")
    sid = lax.axis_index("s")
    wid = sid * NC + cid
    base = wid * ROWS_W

    pltpu.sync_copy(x_hbm.at[pl.ds(base, ROWS_W)], xv)

    def compute_idx(i, carry):
        x = xv[pl.ds(i * 16, 16)]
        f = (x | 1).astype(jnp.float32)
        bits = lax.bitcast_convert_type(f, jnp.int32)
        e = lax.shift_right_logical(bits, 23) - 127
        idx2d[i // 8, pl.ds((i % 8) * 16, 16)] = e
        return carry

    lax.fori_loop(0, ROWS_W // 16, compute_idx, 0, unroll=4)

    def outer(g, carry):
        for b in range(NBUF):
            j = g * NBUF + b

            @pl.when(g > 0)
            def _wait_prev_write():
                pltpu.make_async_copy(
                    bufs.at[b], out_hbm.at[pl.ds(base, CHUNK)], wsems[b]
                ).wait()

            pltpu.async_copy(table_hbm.at[idx2d.at[j]], bufs.at[b], gsem).wait()
            pltpu.async_copy(
                bufs.at[b], out_hbm.at[pl.ds(base + j * CHUNK, CHUNK)], wsems[b]
            )
        return carry

    lax.fori_loop(0, NCHUNK // NBUF, outer, 0)

    for b in range(NBUF):
        pltpu.make_async_copy(
            bufs.at[b], out_hbm.at[pl.ds(base, CHUNK)], wsems[b]
        ).wait()


@jax.jit
def _run(table, xflat):
    mesh = plsc.VectorSubcoreMesh(core_axis_name="c", subcore_axis_name="s")
    scratch = [
        pltpu.VMEM((ROWS_W,), jnp.int32),
        pltpu.VMEM((IDX_ROWS, CHUNK), jnp.int32),
        pltpu.VMEM((NBUF, CHUNK, EMB_DIM), jnp.float32),
        pltpu.SemaphoreType.DMA,
    ] + [pltpu.SemaphoreType.DMA] * NBUF
    k = pl.kernel(
        _body,
        out_type=jax.ShapeDtypeStruct((TOTAL, EMB_DIM), jnp.float32),
        mesh=mesh,
        scratch_types=scratch,
    )
    return k(table, xflat)


def kernel(X, emb_weight):
    xflat = X.astype(jnp.int32).reshape(TOTAL)
    out = _run(emb_weight, xflat)
    return out.reshape(BATCH, 16 * EMB_DIM)


# speedup vs baseline: 19.5018x; 19.5018x over previous
import functools

import jax
import jax.numpy as jnp
from jax import lax
from jax.experimental import pallas as pl
from jax.experimental.pallas import tpu as pltpu
from jax.experimental.pallas import tpu_sc as plsc

BATCH = 16384
EMB_DIM = 128
NUM_EMB = 16

NC = 2
NS = 16
NW = NC * NS
TOTAL = BATCH * 16
ROWS_W = TOTAL // NW
CHUNK = 128
NBUF = 4
NOUTER = ROWS_W // (CHUNK * NBUF)


def _body(tab_hbm, x_hbm, out_hbm, tabv, xv, *bufs_and_sems):
    bufs = bufs_and_sems[:NBUF]
    wsems = bufs_and_sems[NBUF:]
    cid = lax.axis_index("c")
    sid = lax.axis_index("s")
    wid = sid * NC + cid
    base = wid * ROWS_W

    pltpu.sync_copy(tab_hbm, tabv)
    pltpu.sync_copy(x_hbm.at[pl.ds(base, ROWS_W)], xv)

    lane = lax.iota(jnp.int32, 16)
    lane128 = lane * EMB_DIM

    def outer(g, carry):
        for b in range(NBUF):
            j = g * NBUF + b

            @pl.when(g > 0)
            def _wait_prev_write():
                pltpu.make_async_copy(
                    bufs[b], out_hbm.at[pl.ds(0, CHUNK * EMB_DIM)], wsems[b]
                ).wait()

            buf = bufs[b]

            def group(i, carry):
                x = xv[pl.ds(j * CHUNK + i * 16, 16)]
                f = (x | 1).astype(jnp.float32)
                bits = lax.bitcast_convert_type(f, jnp.int32)
                e128 = (lax.shift_right_logical(bits, 16) - (127 << 7)) & -128
                rowb = lane128 + i * (16 * EMB_DIM)

                @plsc.parallel_loop(0, EMB_DIM, unroll=8)
                def phase(c0):
                    cols = (lane + c0) & (EMB_DIM - 1)
                    vals = plsc.load_gather(tabv, [e128 + cols])
                    plsc.store_scatter(buf, [rowb + cols], vals)

                return carry

            lax.fori_loop(0, CHUNK // 16, group, 0)

            pltpu.async_copy(
                buf,
                out_hbm.at[pl.ds((base + j * CHUNK) * EMB_DIM, CHUNK * EMB_DIM)],
                wsems[b],
            )
        return carry

    lax.fori_loop(0, NOUTER, outer, 0)

    for b in range(NBUF):
        pltpu.make_async_copy(
            bufs[b], out_hbm.at[pl.ds(0, CHUNK * EMB_DIM)], wsems[b]
        ).wait()


@jax.jit
def _run(tabflat, xflat):
    mesh = plsc.VectorSubcoreMesh(core_axis_name="c", subcore_axis_name="s")
    scratch = [
        pltpu.VMEM((NUM_EMB * EMB_DIM,), jnp.float32),
        pltpu.VMEM((ROWS_W,), jnp.int32),
    ] + [pltpu.VMEM((CHUNK * EMB_DIM,), jnp.float32)] * NBUF + [
        pltpu.SemaphoreType.DMA
    ] * NBUF
    k = pl.kernel(
        _body,
        out_type=jax.ShapeDtypeStruct((TOTAL * EMB_DIM,), jnp.float32),
        mesh=mesh,
        scratch_types=scratch,
        compiler_params=pltpu.CompilerParams(needs_layout_passes=False),
    )
    return k(tabflat, xflat)


def kernel(X, emb_weight):
    xflat = X.astype(jnp.int32).reshape(TOTAL)
    tabflat = emb_weight.reshape(NUM_EMB * EMB_DIM)
    out = _run(tabflat, xflat)
    return out.reshape(BATCH, 16 * EMB_DIM)
